# baseline (speedup 1.0000x reference)
import jax
import jax.numpy as jnp
from jax.experimental import pallas as pl

_BV, _LS, _C = 32, 196, 512
_BT, _LW = 32, 50
_H, _K = 102, 47
_NKEEP = 118
_NEG = -1e30
_BP = 16
_BM = 4
_NBP = _BV // _BP
_NBM = _BV // _BM


def _rownorm(x):
    n = jnp.sqrt(jnp.sum(x * x, axis=-1, keepdims=True))
    return x / jnp.maximum(n, 1e-12)


def _precompute_kernel(cls_ref, sp_ref, g_ref, b_ref, w1_ref, b1_ref,
                       w2_ref, b2_ref, scale_ref,
                       sn_ref, sa_ref, glo_ref, wlt_ref):
    cls = cls_ref[...]
    sp = sp_ref[...]
    glo = _rownorm(cls)
    sn = _rownorm(sp)
    sa = jnp.sum(glo[:, None, :] * sn, axis=-1)

    m = jnp.mean(sp, axis=-1, keepdims=True)
    v = jnp.mean((sp - m) ** 2, axis=-1, keepdims=True)
    ln = (sp - m) / jnp.sqrt(v + 1e-5) * g_ref[0] + b_ref[0]
    ln2 = ln.reshape(_BP * _LS, _C)

    h = jnp.dot(ln2, w1_ref[...], preferred_element_type=jnp.float32)
    h = h + b1_ref[0]
    h = 0.5 * h * (1.0 + jax.lax.erf(h / jnp.sqrt(2.0).astype(jnp.float32)))
    wl = jnp.dot(h, w2_ref[...], preferred_element_type=jnp.float32)
    wl = (wl + b2_ref[0]) * scale_ref[0, 0]
    wl = wl.reshape(_BP, _LS, _K)

    sn_ref[...] = sn
    sa_ref[...] = sa
    glo_ref[...] = glo
    wlt_ref[...] = jnp.transpose(wl, (0, 2, 1))


def _main_kernel(sp_ref, sn_ref, sa_ref, wlt_ref, glo_ref, cap_ref, vm_ref,
                 out_ref):
    sp = sp_ref[...]
    sn = sn_ref[...]
    sa = sa_ref[0]
    wlt = wlt_ref[...]
    glo = glo_ref[0]
    capn = _rownorm(cap_ref[0])
    vm = vm_ref[0]

    cap_glo = capn[0]
    cap_attn = jnp.sum(cap_glo[None, None, :] * sn, axis=-1)
    score = sa + cap_attn

    s_j = score[:, :, None]
    s_l = score[:, None, :]
    il = jax.lax.broadcasted_iota(jnp.int32, (1, _LS, _LS), 2)
    ij = jax.lax.broadcasted_iota(jnp.int32, (1, _LS, _LS), 1)
    before = (s_l > s_j) | ((s_l == s_j) & (il < ij))
    rank = jnp.sum(before.astype(jnp.float32), axis=2)
    keep = rank < float(_NKEEP)

    lg_k = jnp.where(keep[:, None, :], wlt, _NEG)
    lg_x = jnp.where(keep, _NEG, score)[:, None, :]
    lg = jnp.concatenate([lg_k, lg_x], axis=1)
    e = jnp.exp(lg - jnp.max(lg, axis=-1, keepdims=True))
    w = e / jnp.sum(e, axis=-1, keepdims=True)

    aggr = jax.lax.dot_general(w, sp, (((2,), (1,)), ((0,), (0,))),
                               preferred_element_type=jnp.float32)
    aggr = _rownorm(aggr)

    cnt = jnp.transpose(capn, (1, 0))
    sim_g = jnp.dot(glo, cnt, preferred_element_type=jnp.float32)
    sim_a = jax.lax.dot_general(aggr, cnt, (((2,), (0,)), ((), ())),
                                preferred_element_type=jnp.float32)
    mx = jnp.maximum(jnp.max(sim_a, axis=1), sim_g)
    out_ref[0, 0, :] = jnp.sum(mx * vm, axis=1)


def kernel(img_embs, cap_embs, cap_lens, ln_g, ln_b, w1, b1, w2, b2, scale):
    f32 = jnp.float32
    cls = img_embs[:, 0, :]
    spatial = img_embs[:, 1:, :]

    sn, sa, glo, wlt = pl.pallas_call(
        _precompute_kernel,
        grid=(_NBP,),
        in_specs=[
            pl.BlockSpec((_BP, _C), lambda j: (j, 0)),
            pl.BlockSpec((_BP, _LS, _C), lambda j: (j, 0, 0)),
            pl.BlockSpec((1, _C), lambda j: (0, 0)),
            pl.BlockSpec((1, _C), lambda j: (0, 0)),
            pl.BlockSpec((_C, _H), lambda j: (0, 0)),
            pl.BlockSpec((1, _H), lambda j: (0, 0)),
            pl.BlockSpec((_H, _K), lambda j: (0, 0)),
            pl.BlockSpec((1, _K), lambda j: (0, 0)),
            pl.BlockSpec((1, 1), lambda j: (0, 0)),
        ],
        out_specs=(
            pl.BlockSpec((_BP, _LS, _C), lambda j: (j, 0, 0)),
            pl.BlockSpec((_BP, _LS), lambda j: (j, 0)),
            pl.BlockSpec((_BP, _C), lambda j: (j, 0)),
            pl.BlockSpec((_BP, _K, _LS), lambda j: (j, 0, 0)),
        ),
        out_shape=(
            jax.ShapeDtypeStruct((_BV, _LS, _C), f32),
            jax.ShapeDtypeStruct((_BV, _LS), f32),
            jax.ShapeDtypeStruct((_BV, _C), f32),
            jax.ShapeDtypeStruct((_BV, _K, _LS), f32),
        ),
    )(cls, spatial, ln_g.reshape(1, _C), ln_b.reshape(1, _C),
      w1, b1.reshape(1, _H), w2, b2.reshape(1, _K), scale.reshape(1, 1))

    sa3 = sa.reshape(_NBM, _BM, _LS)
    glo3 = glo.reshape(_NBM, _BM, _C)

    lens = cap_lens.astype(f32)[:, None]
    vm = ((jnp.arange(_LW)[None, :] < cap_lens[:, None]).astype(f32)
          / lens).reshape(_BT, 1, _LW)

    out = pl.pallas_call(
        _main_kernel,
        grid=(_BT, _NBM),
        in_specs=[
            pl.BlockSpec((_BM, _LS, _C), lambda i, j: (j, 0, 0)),
            pl.BlockSpec((_BM, _LS, _C), lambda i, j: (j, 0, 0)),
            pl.BlockSpec((1, _BM, _LS), lambda i, j: (j, 0, 0)),
            pl.BlockSpec((_BM, _K, _LS), lambda i, j: (j, 0, 0)),
            pl.BlockSpec((1, _BM, _C), lambda i, j: (j, 0, 0)),
            pl.BlockSpec((1, _LW, _C), lambda i, j: (i, 0, 0)),
            pl.BlockSpec((1, 1, _LW), lambda i, j: (i, 0, 0)),
        ],
        out_specs=pl.BlockSpec((1, 1, _BM), lambda i, j: (i * _NBM + j, 0, 0)),
        out_shape=jax.ShapeDtypeStruct((_BT * _NBM, 1, _BM), f32),
    )(spatial, sn, sa3, wlt, glo3, cap_embs, vm)

    return out.reshape(_BT, _BV).T


# speedup vs baseline: 1.0010x; 1.0010x over previous
import jax
import jax.numpy as jnp
from jax.experimental import pallas as pl

_BV, _LS, _C = 32, 196, 512
_BT, _LW = 32, 50
_H, _K = 102, 47
_NKEEP = 118
_NEG = -1e30
_BP = 16
_BM = 4
_NBP = _BV // _BP
_NBM = _BV // _BM


def _rownorm(x):
    n = jnp.sqrt(jnp.sum(x * x, axis=-1, keepdims=True))
    return x / jnp.maximum(n, 1e-12)


def _precompute_kernel(cls_ref, sp_ref, g_ref, b_ref, w1_ref, b1_ref,
                       w2_ref, b2_ref, scale_ref,
                       sn_ref, sa_ref, glo_ref, wlt_ref):
    cls = cls_ref[...]
    sp = sp_ref[...]
    glo = _rownorm(cls)
    sn = _rownorm(sp)
    sa = jnp.sum(glo[:, None, :] * sn, axis=-1)

    m = jnp.mean(sp, axis=-1, keepdims=True)
    v = jnp.mean((sp - m) ** 2, axis=-1, keepdims=True)
    ln = (sp - m) / jnp.sqrt(v + 1e-5) * g_ref[0] + b_ref[0]
    ln2 = ln.reshape(_BP * _LS, _C)

    h = jnp.dot(ln2, w1_ref[...], preferred_element_type=jnp.float32)
    h = h + b1_ref[0]
    h = 0.5 * h * (1.0 + jax.lax.erf(h / jnp.sqrt(2.0).astype(jnp.float32)))
    wl = jnp.dot(h, w2_ref[...], preferred_element_type=jnp.float32)
    wl = (wl + b2_ref[0]) * scale_ref[0, 0]
    wl = wl.reshape(_BP, _LS, _K)

    sn_ref[...] = sn
    sa_ref[...] = sa
    glo_ref[...] = glo
    wlt_ref[...] = jnp.transpose(wl, (0, 2, 1))


def _main_kernel(sp_ref, sn_ref, sa_ref, wlt_ref, glo_ref, cap_ref, vm_ref,
                 out_ref):
    sp = sp_ref[...]
    sn = sn_ref[...]
    sa = sa_ref[0]
    wlt = wlt_ref[...]
    glo = glo_ref[0]
    capn = _rownorm(cap_ref[0])
    vm = vm_ref[0]

    cap_glo = capn[0]
    cap_attn = jnp.sum(cap_glo[None, None, :] * sn, axis=-1)
    score = sa + cap_attn

    s_j = score[:, :, None]
    s_l = score[:, None, :]
    il = jax.lax.broadcasted_iota(jnp.int32, (1, _LS, _LS), 2)
    ij = jax.lax.broadcasted_iota(jnp.int32, (1, _LS, _LS), 1)
    before = (s_l > s_j) | ((s_l == s_j) & (il < ij))
    rank = jnp.sum(before.astype(jnp.float32), axis=2)
    keep = rank < float(_NKEEP)

    lg_k = jnp.where(keep[:, None, :], wlt, _NEG)
    lg_x = jnp.where(keep, _NEG, score)[:, None, :]
    lg = jnp.concatenate([lg_k, lg_x], axis=1)
    e = jnp.exp(lg - jnp.max(lg, axis=-1, keepdims=True))
    w = e / jnp.sum(e, axis=-1, keepdims=True)

    aggr = jax.lax.dot_general(w, sp, (((2,), (1,)), ((0,), (0,))),
                               preferred_element_type=jnp.float32)
    aggr = _rownorm(aggr)

    cnt = jnp.transpose(capn, (1, 0))
    sim_g = jnp.dot(glo, cnt, preferred_element_type=jnp.float32)
    sim_a = jax.lax.dot_general(aggr, cnt, (((2,), (0,)), ((), ())),
                                preferred_element_type=jnp.float32)
    mx = jnp.maximum(jnp.max(sim_a, axis=1), sim_g)
    out_ref[0, 0, :] = jnp.sum(mx * vm, axis=1)


def kernel(img_embs, cap_embs, cap_lens, ln_g, ln_b, w1, b1, w2, b2, scale):
    f32 = jnp.float32
    cls = img_embs[:, 0, :]
    spatial = img_embs[:, 1:, :]

    sn, sa, glo, wlt = pl.pallas_call(
        _precompute_kernel,
        grid=(_NBP,),
        in_specs=[
            pl.BlockSpec((_BP, _C), lambda j: (j, 0)),
            pl.BlockSpec((_BP, _LS, _C), lambda j: (j, 0, 0)),
            pl.BlockSpec((1, _C), lambda j: (0, 0)),
            pl.BlockSpec((1, _C), lambda j: (0, 0)),
            pl.BlockSpec((_C, _H), lambda j: (0, 0)),
            pl.BlockSpec((1, _H), lambda j: (0, 0)),
            pl.BlockSpec((_H, _K), lambda j: (0, 0)),
            pl.BlockSpec((1, _K), lambda j: (0, 0)),
            pl.BlockSpec((1, 1), lambda j: (0, 0)),
        ],
        out_specs=(
            pl.BlockSpec((_BP, _LS, _C), lambda j: (j, 0, 0)),
            pl.BlockSpec((_BP, _LS), lambda j: (j, 0)),
            pl.BlockSpec((_BP, _C), lambda j: (j, 0)),
            pl.BlockSpec((_BP, _K, _LS), lambda j: (j, 0, 0)),
        ),
        out_shape=(
            jax.ShapeDtypeStruct((_BV, _LS, _C), f32),
            jax.ShapeDtypeStruct((_BV, _LS), f32),
            jax.ShapeDtypeStruct((_BV, _C), f32),
            jax.ShapeDtypeStruct((_BV, _K, _LS), f32),
        ),
    )(cls, spatial, ln_g.reshape(1, _C), ln_b.reshape(1, _C),
      w1, b1.reshape(1, _H), w2, b2.reshape(1, _K), scale.reshape(1, 1))

    sa3 = sa.reshape(_NBM, _BM, _LS)
    glo3 = glo.reshape(_NBM, _BM, _C)

    lens = cap_lens.astype(f32)[:, None]
    vm = ((jnp.arange(_LW)[None, :] < cap_lens[:, None]).astype(f32)
          / lens).reshape(_BT, 1, _LW)

    out = pl.pallas_call(
        _main_kernel,
        grid=(_NBM, _BT),
        in_specs=[
            pl.BlockSpec((_BM, _LS, _C), lambda j, i: (j, 0, 0)),
            pl.BlockSpec((_BM, _LS, _C), lambda j, i: (j, 0, 0)),
            pl.BlockSpec((1, _BM, _LS), lambda j, i: (j, 0, 0)),
            pl.BlockSpec((_BM, _K, _LS), lambda j, i: (j, 0, 0)),
            pl.BlockSpec((1, _BM, _C), lambda j, i: (j, 0, 0)),
            pl.BlockSpec((1, _LW, _C), lambda j, i: (i, 0, 0)),
            pl.BlockSpec((1, 1, _LW), lambda j, i: (i, 0, 0)),
        ],
        out_specs=pl.BlockSpec((1, 1, _BM), lambda j, i: (j * _BT + i, 0, 0)),
        out_shape=jax.ShapeDtypeStruct((_NBM * _BT, 1, _BM), f32),
    )(spatial, sn, sa3, wlt, glo3, cap_embs, vm)

    return out.reshape(_NBM, _BT, _BM).transpose(0, 2, 1).reshape(_BV, _BT)


# speedup vs baseline: 4.9499x; 4.9450x over previous
import jax
import jax.numpy as jnp
from jax.experimental import pallas as pl

_BV, _LS, _C = 32, 196, 512
_BT, _LW = 32, 50
_H, _K = 102, 47
_NKEEP = 118
_NEG = -1e30
_BP = 16
_BM = 8
_NBP = _BV // _BP
_NBM = _BV // _BM


def _rownorm(x):
    n = jnp.sqrt(jnp.sum(x * x, axis=-1, keepdims=True))
    return x / jnp.maximum(n, 1e-12)


def _precompute_kernel(cls_ref, sp_ref, g_ref, b_ref, w1_ref, b1_ref,
                       w2_ref, b2_ref, scale_ref,
                       sn_ref, sa_ref, glo_ref, wlt_ref):
    cls = cls_ref[...]
    sp = sp_ref[...]
    glo = _rownorm(cls)
    sn = _rownorm(sp)
    sa = jnp.sum(glo[:, None, :] * sn, axis=-1)

    m = jnp.mean(sp, axis=-1, keepdims=True)
    v = jnp.mean((sp - m) ** 2, axis=-1, keepdims=True)
    ln = (sp - m) / jnp.sqrt(v + 1e-5) * g_ref[0] + b_ref[0]
    ln2 = ln.reshape(_BP * _LS, _C)

    h = jnp.dot(ln2, w1_ref[...], preferred_element_type=jnp.float32)
    h = h + b1_ref[0]
    h = 0.5 * h * (1.0 + jax.lax.erf(h / jnp.sqrt(2.0).astype(jnp.float32)))
    wl = jnp.dot(h, w2_ref[...], preferred_element_type=jnp.float32)
    wl = (wl + b2_ref[0]) * scale_ref[0, 0]
    wl = wl.reshape(_BP, _LS, _K)

    sn_ref[...] = sn
    sa_ref[...] = sa
    glo_ref[...] = glo
    wlt_ref[...] = jnp.transpose(wl, (0, 2, 1))


def _main_kernel(sp_ref, sn_ref, sa_ref, wlt_ref, glo_ref, cap_ref, vm_ref,
                 out_ref):
    sp = sp_ref[...]
    sn = sn_ref[...]
    sa = sa_ref[0]
    wlt = wlt_ref[...]
    glo = glo_ref[0]
    capn = _rownorm(cap_ref[0])
    vm = vm_ref[0]

    cap_glo = capn[0]
    cap_attn = jnp.sum(cap_glo[None, None, :] * sn, axis=-1)
    score = sa + cap_attn

    b32 = jax.lax.bitcast_convert_type(score, jnp.int32)
    keys = jnp.where(b32 >= 0, b32, b32 ^ jnp.int32(0x7FFFFFFF))
    lo0 = jnp.min(keys, axis=1, keepdims=True)
    hi0 = jnp.max(keys, axis=1, keepdims=True)

    def _bs_body(_, carry):
        lo, hi = carry
        live = lo < hi
        mid = (lo >> 1) + (hi >> 1) + (lo & hi & 1)
        cnt = jnp.sum((keys > mid).astype(jnp.int32), axis=1, keepdims=True)
        p = cnt < _NKEEP
        lo2 = jnp.where(p, lo, mid + 1)
        hi2 = jnp.where(p, mid, hi)
        return (jnp.where(live, lo2, lo), jnp.where(live, hi2, hi))

    t_key, _ = jax.lax.fori_loop(0, 32, _bs_body, (lo0, hi0))
    t_val = jax.lax.bitcast_convert_type(
        jnp.where(t_key >= 0, t_key, t_key ^ jnp.int32(0x7FFFFFFF)),
        jnp.float32)

    gt = score > t_val
    eq = score == t_val
    need = (jnp.float32(_NKEEP)
            - jnp.sum(gt.astype(jnp.float32), axis=1, keepdims=True))
    il = jax.lax.broadcasted_iota(jnp.int32, (_LS, _LS), 0)
    ij = jax.lax.broadcasted_iota(jnp.int32, (_LS, _LS), 1)
    strict_lt = (il < ij).astype(jnp.float32)
    cum = jnp.dot(eq.astype(jnp.float32), strict_lt,
                  preferred_element_type=jnp.float32)
    keep = gt | (eq & (cum < need))

    lg_k = jnp.where(keep[:, None, :], wlt, _NEG)
    lg_x = jnp.where(keep, _NEG, score)[:, None, :]
    lg = jnp.concatenate([lg_k, lg_x], axis=1)
    e = jnp.exp(lg - jnp.max(lg, axis=-1, keepdims=True))
    w = e / jnp.sum(e, axis=-1, keepdims=True)

    aggr = jax.lax.dot_general(w, sp, (((2,), (1,)), ((0,), (0,))),
                               preferred_element_type=jnp.float32)
    aggr = _rownorm(aggr)

    cnt = jnp.transpose(capn, (1, 0))
    sim_g = jnp.dot(glo, cnt, preferred_element_type=jnp.float32)
    sim_a = jax.lax.dot_general(aggr, cnt, (((2,), (0,)), ((), ())),
                                preferred_element_type=jnp.float32)
    mx = jnp.maximum(jnp.max(sim_a, axis=1), sim_g)
    out_ref[0, 0, :] = jnp.sum(mx * vm, axis=1)


def kernel(img_embs, cap_embs, cap_lens, ln_g, ln_b, w1, b1, w2, b2, scale):
    f32 = jnp.float32
    cls = img_embs[:, 0, :]
    spatial = img_embs[:, 1:, :]

    sn, sa, glo, wlt = pl.pallas_call(
        _precompute_kernel,
        grid=(_NBP,),
        in_specs=[
            pl.BlockSpec((_BP, _C), lambda j: (j, 0)),
            pl.BlockSpec((_BP, _LS, _C), lambda j: (j, 0, 0)),
            pl.BlockSpec((1, _C), lambda j: (0, 0)),
            pl.BlockSpec((1, _C), lambda j: (0, 0)),
            pl.BlockSpec((_C, _H), lambda j: (0, 0)),
            pl.BlockSpec((1, _H), lambda j: (0, 0)),
            pl.BlockSpec((_H, _K), lambda j: (0, 0)),
            pl.BlockSpec((1, _K), lambda j: (0, 0)),
            pl.BlockSpec((1, 1), lambda j: (0, 0)),
        ],
        out_specs=(
            pl.BlockSpec((_BP, _LS, _C), lambda j: (j, 0, 0)),
            pl.BlockSpec((_BP, _LS), lambda j: (j, 0)),
            pl.BlockSpec((_BP, _C), lambda j: (j, 0)),
            pl.BlockSpec((_BP, _K, _LS), lambda j: (j, 0, 0)),
        ),
        out_shape=(
            jax.ShapeDtypeStruct((_BV, _LS, _C), f32),
            jax.ShapeDtypeStruct((_BV, _LS), f32),
            jax.ShapeDtypeStruct((_BV, _C), f32),
            jax.ShapeDtypeStruct((_BV, _K, _LS), f32),
        ),
    )(cls, spatial, ln_g.reshape(1, _C), ln_b.reshape(1, _C),
      w1, b1.reshape(1, _H), w2, b2.reshape(1, _K), scale.reshape(1, 1))

    sa3 = sa.reshape(_NBM, _BM, _LS)
    glo3 = glo.reshape(_NBM, _BM, _C)

    lens = cap_lens.astype(f32)[:, None]
    vm = ((jnp.arange(_LW)[None, :] < cap_lens[:, None]).astype(f32)
          / lens).reshape(_BT, 1, _LW)

    out = pl.pallas_call(
        _main_kernel,
        grid=(_NBM, _BT),
        in_specs=[
            pl.BlockSpec((_BM, _LS, _C), lambda j, i: (j, 0, 0)),
            pl.BlockSpec((_BM, _LS, _C), lambda j, i: (j, 0, 0)),
            pl.BlockSpec((1, _BM, _LS), lambda j, i: (j, 0, 0)),
            pl.BlockSpec((_BM, _K, _LS), lambda j, i: (j, 0, 0)),
            pl.BlockSpec((1, _BM, _C), lambda j, i: (j, 0, 0)),
            pl.BlockSpec((1, _LW, _C), lambda j, i: (i, 0, 0)),
            pl.BlockSpec((1, 1, _LW), lambda j, i: (i, 0, 0)),
        ],
        out_specs=pl.BlockSpec((1, 1, _BM), lambda j, i: (j * _BT + i, 0, 0)),
        out_shape=jax.ShapeDtypeStruct((_NBM * _BT, 1, _BM), f32),
    )(spatial, sn, sa3, wlt, glo3, cap_embs, vm)

    return out.reshape(_NBM, _BT, _BM).transpose(0, 2, 1).reshape(_BV, _BT)


# speedup vs baseline: 8.0455x; 1.6254x over previous
import jax
import jax.numpy as jnp
from jax.experimental import pallas as pl

_BV, _LS, _C = 32, 197 - 1, 512
_BT, _LW = 32, 50
_H, _K = 102, 47
_NKEEP = 118
_NEG = -1e30
_BP = 16
_BM = 16
_NBP = _BV // _BP
_NBM = _BV // _BM


def _rownorm(x):
    n = jnp.sqrt(jnp.sum(x * x, axis=-1, keepdims=True))
    return x / jnp.maximum(n, 1e-12)


def _precompute_kernel(cls_ref, sp_ref, capcls_ref, g_ref, b_ref,
                       w1_ref, b1_ref, w2_ref, b2_ref, scale_ref,
                       ca_ref, sa_ref, glo_ref, wlt_ref):
    cls = cls_ref[...]
    sp = sp_ref[...]
    glo = _rownorm(cls)
    sn = _rownorm(sp)
    sa = jnp.sum(glo[:, None, :] * sn, axis=-1)

    capg = _rownorm(capcls_ref[...])
    ca = jax.lax.dot_general(
        sn.reshape(_BP * _LS, _C), jnp.transpose(capg, (1, 0)),
        (((1,), (0,)), ((), ())),
        precision=jax.lax.Precision.HIGHEST,
        preferred_element_type=jnp.float32)
    ca_ref[...] = ca.reshape(_BP, _LS, _BT)

    m = jnp.mean(sp, axis=-1, keepdims=True)
    v = jnp.mean((sp - m) ** 2, axis=-1, keepdims=True)
    ln = (sp - m) / jnp.sqrt(v + 1e-5) * g_ref[0] + b_ref[0]
    ln2 = ln.reshape(_BP * _LS, _C)

    h = jnp.dot(ln2, w1_ref[...], preferred_element_type=jnp.float32)
    h = h + b1_ref[0]
    h = 0.5 * h * (1.0 + jax.lax.erf(h / jnp.sqrt(2.0).astype(jnp.float32)))
    wl = jnp.dot(h, w2_ref[...], preferred_element_type=jnp.float32)
    wl = (wl + b2_ref[0]) * scale_ref[0, 0]
    wl = wl.reshape(_BP, _LS, _K)

    sa_ref[...] = sa
    glo_ref[...] = glo
    wlt_ref[...] = jnp.transpose(wl, (0, 2, 1))


def _main_kernel(sp_ref, ca_ref, sa_ref, wlt_ref, glo_ref, cap_ref, vm_ref,
                 out_ref):
    sp = sp_ref[...]
    ca = ca_ref[...]
    sa = sa_ref[0]
    wlt = wlt_ref[...]
    glo = glo_ref[0]
    capn = _rownorm(cap_ref[0])
    vm = vm_ref[0]

    i = pl.program_id(1)
    onehot = (jax.lax.broadcasted_iota(jnp.int32, (1, 1, _BT), 2) == i
              ).astype(jnp.float32)
    score = sa + jnp.sum(ca * onehot, axis=-1)

    b32 = jax.lax.bitcast_convert_type(score, jnp.int32)
    keys = jnp.where(b32 >= 0, b32, b32 ^ jnp.int32(0x7FFFFFFF))
    lo0 = jnp.min(keys, axis=1, keepdims=True)
    hi0 = jnp.max(keys, axis=1, keepdims=True)

    def _bs_body(_, carry):
        lo, hi = carry
        live = lo < hi
        mid = (lo >> 1) + (hi >> 1) + (lo & hi & 1)
        cnt = jnp.sum((keys > mid).astype(jnp.int32), axis=1, keepdims=True)
        p = cnt < _NKEEP
        lo2 = jnp.where(p, lo, mid + 1)
        hi2 = jnp.where(p, mid, hi)
        return (jnp.where(live, lo2, lo), jnp.where(live, hi2, hi))

    t_key, _ = jax.lax.fori_loop(0, 32, _bs_body, (lo0, hi0))
    t_val = jax.lax.bitcast_convert_type(
        jnp.where(t_key >= 0, t_key, t_key ^ jnp.int32(0x7FFFFFFF)),
        jnp.float32)

    gt = score > t_val
    eq = score == t_val
    need = (jnp.float32(_NKEEP)
            - jnp.sum(gt.astype(jnp.float32), axis=1, keepdims=True))
    il = jax.lax.broadcasted_iota(jnp.int32, (_LS, _LS), 0)
    ij = jax.lax.broadcasted_iota(jnp.int32, (_LS, _LS), 1)
    strict_lt = (il < ij).astype(jnp.float32)
    cum = jnp.dot(eq.astype(jnp.float32), strict_lt,
                  preferred_element_type=jnp.float32)
    keep = gt | (eq & (cum < need))

    lg_k = jnp.where(keep[:, None, :], wlt, _NEG)
    lg_x = jnp.where(keep, _NEG, score)[:, None, :]
    lg = jnp.concatenate([lg_k, lg_x], axis=1)
    e = jnp.exp(lg - jnp.max(lg, axis=-1, keepdims=True))
    w = e / jnp.sum(e, axis=-1, keepdims=True)

    aggr = jax.lax.dot_general(w, sp, (((2,), (1,)), ((0,), (0,))),
                               preferred_element_type=jnp.float32)
    aggr = _rownorm(aggr)

    cnt = jnp.transpose(capn, (1, 0))
    sim_g = jnp.dot(glo, cnt, preferred_element_type=jnp.float32)
    sim_a = jax.lax.dot_general(aggr, cnt, (((2,), (0,)), ((), ())),
                                preferred_element_type=jnp.float32)
    mx = jnp.maximum(jnp.max(sim_a, axis=1), sim_g)
    out_ref[0, 0, :] = jnp.sum(mx * vm, axis=1)


def kernel(img_embs, cap_embs, cap_lens, ln_g, ln_b, w1, b1, w2, b2, scale):
    f32 = jnp.float32
    cls = img_embs[:, 0, :]
    spatial = img_embs[:, 1:, :]
    cap_cls = cap_embs[:, 0, :]

    ca, sa, glo, wlt = pl.pallas_call(
        _precompute_kernel,
        grid=(_NBP,),
        in_specs=[
            pl.BlockSpec((_BP, _C), lambda j: (j, 0)),
            pl.BlockSpec((_BP, _LS, _C), lambda j: (j, 0, 0)),
            pl.BlockSpec((_BT, _C), lambda j: (0, 0)),
            pl.BlockSpec((1, _C), lambda j: (0, 0)),
            pl.BlockSpec((1, _C), lambda j: (0, 0)),
            pl.BlockSpec((_C, _H), lambda j: (0, 0)),
            pl.BlockSpec((1, _H), lambda j: (0, 0)),
            pl.BlockSpec((_H, _K), lambda j: (0, 0)),
            pl.BlockSpec((1, _K), lambda j: (0, 0)),
            pl.BlockSpec((1, 1), lambda j: (0, 0)),
        ],
        out_specs=(
            pl.BlockSpec((_BP, _LS, _BT), lambda j: (j, 0, 0)),
            pl.BlockSpec((_BP, _LS), lambda j: (j, 0)),
            pl.BlockSpec((_BP, _C), lambda j: (j, 0)),
            pl.BlockSpec((_BP, _K, _LS), lambda j: (j, 0, 0)),
        ),
        out_shape=(
            jax.ShapeDtypeStruct((_BV, _LS, _BT), f32),
            jax.ShapeDtypeStruct((_BV, _LS), f32),
            jax.ShapeDtypeStruct((_BV, _C), f32),
            jax.ShapeDtypeStruct((_BV, _K, _LS), f32),
        ),
    )(cls, spatial, cap_cls, ln_g.reshape(1, _C), ln_b.reshape(1, _C),
      w1, b1.reshape(1, _H), w2, b2.reshape(1, _K), scale.reshape(1, 1))

    sa3 = sa.reshape(_NBM, _BM, _LS)
    glo3 = glo.reshape(_NBM, _BM, _C)

    lens = cap_lens.astype(f32)[:, None]
    vm = ((jnp.arange(_LW)[None, :] < cap_lens[:, None]).astype(f32)
          / lens).reshape(_BT, 1, _LW)

    out = pl.pallas_call(
        _main_kernel,
        grid=(_NBM, _BT),
        in_specs=[
            pl.BlockSpec((_BM, _LS, _C), lambda j, i: (j, 0, 0)),
            pl.BlockSpec((_BM, _LS, _BT), lambda j, i: (j, 0, 0)),
            pl.BlockSpec((1, _BM, _LS), lambda j, i: (j, 0, 0)),
            pl.BlockSpec((_BM, _K, _LS), lambda j, i: (j, 0, 0)),
            pl.BlockSpec((1, _BM, _C), lambda j, i: (j, 0, 0)),
            pl.BlockSpec((1, _LW, _C), lambda j, i: (i, 0, 0)),
            pl.BlockSpec((1, 1, _LW), lambda j, i: (i, 0, 0)),
        ],
        out_specs=pl.BlockSpec((1, 1, _BM), lambda j, i: (j * _BT + i, 0, 0)),
        out_shape=jax.ShapeDtypeStruct((_NBM * _BT, 1, _BM), f32),
    )(spatial, ca, sa3, wlt, glo3, cap_embs, vm)

    return out.reshape(_NBM, _BT, _BM).transpose(0, 2, 1).reshape(_BV, _BT)
